# baseline (speedup 1.0000x reference)
import functools

import jax
import jax.numpy as jnp
from jax import lax
from jax.experimental import pallas as pl
from jax.experimental.pallas import tpu as pltpu
from jax.experimental.pallas import tpu_sc as plsc

NC, NS = 2, 16
NW = NC * NS
ROWS = 16384
N = 1024
RPW = ROWS // NW
G = 16
NGRP = RPW // G


def _vsqrt(x):
    i = lax.bitcast_convert_type(x, jnp.int32)
    y = lax.bitcast_convert_type((i >> 1) + jnp.int32(0x1FBD1DF6), jnp.float32)
    for _ in range(3):
        y = 0.5 * (y + x / y)
    return y


def _sort_rows(S0, S1):
    def row_body(j, _):
        def p0(i, _):
            v = jnp.abs(S0[j, pl.ds(i * 16, 16)])
            S0[j, pl.ds(i * 16, 16)] = lax.sort(v, dimension=0, is_stable=False)
            return 0

        lax.fori_loop(0, N // 16, p0, 0)

        refs = (S0, S1)
        cur = 0
        for m in (16, 32, 64, 128, 256, 512):
            src, dst = refs[cur], refs[1 - cur]

            def merge_body(mi, _, m=m, src=src, dst=dst):
                base = mi * (2 * m)

                def s1(jj, _):
                    va = src[j, pl.ds(base + jj * 16, 16)]
                    vb = lax.rev(
                        src[j, pl.ds(base + 2 * m - (jj + 1) * 16, 16)], (0,)
                    )
                    dst[j, pl.ds(base + jj * 16, 16)] = jnp.minimum(va, vb)
                    dst[j, pl.ds(base + m + jj * 16, 16)] = jnp.maximum(va, vb)
                    return 0

                lax.fori_loop(0, m // 16, s1, 0)

                d = m // 2
                while d >= 16:
                    def s2(t, _, d=d):
                        b = t // (d // 16)
                        jj = t % (d // 16)
                        off = base + b * 2 * d + jj * 16
                        xlo = dst[j, pl.ds(off, 16)]
                        xhi = dst[j, pl.ds(off + d, 16)]
                        dst[j, pl.ds(off, 16)] = jnp.minimum(xlo, xhi)
                        dst[j, pl.ds(off + d, 16)] = jnp.maximum(xlo, xhi)
                        return 0

                    lax.fori_loop(0, ((2 * m) // (2 * d)) * (d // 16), s2, 0)
                    d //= 2

                def s3(jj, _):
                    off = base + jj * 16
                    dst[j, pl.ds(off, 16)] = lax.sort(
                        dst[j, pl.ds(off, 16)], dimension=0, is_stable=False
                    )
                    return 0

                lax.fori_loop(0, (2 * m) // 16, s3, 0)
                return 0

            lax.fori_loop(0, 512 // m, merge_body, 0)
            cur = 1 - cur
        return 0

    lax.fori_loop(0, G, row_body, 0)


def _row_sums(S0, T0):
    rows = lax.iota(jnp.int32, 16)

    def dbody(p, acc):
        col = jnp.full((16,), p, dtype=jnp.int32)
        av = plsc.load_gather(S0, [rows, col])
        bv = plsc.load_gather(T0, [rows, col])
        dd = av - bv
        return acc + dd * dd

    return lax.fori_loop(0, N, dbody, jnp.zeros((16,), jnp.float32))


_MESH = plsc.VectorSubcoreMesh(
    core_axis_name="c", subcore_axis_name="s", num_cores=NC, num_subcores=NS
)


@functools.partial(
    pl.kernel,
    mesh=_MESH,
    out_type=jax.ShapeDtypeStruct((NW, G), jnp.float32),
    compiler_params=pltpu.CompilerParams(needs_layout_passes=False),
    scratch_types=[
        pltpu.VMEM((G, N), jnp.float32),
        pltpu.VMEM((G, N), jnp.float32),
        pltpu.VMEM((G, N), jnp.float32),
        pltpu.VMEM((G, N), jnp.float32),
        pltpu.VMEM((G,), jnp.float32),
    ],
)
def _sc_loss(a_hbm, b_hbm, out_hbm, S0, S1, T0, T1, accv):
    wid = lax.axis_index("s") * NC + lax.axis_index("c")

    def group(t, loss):
        base = wid * RPW + t * G
        pltpu.sync_copy(a_hbm.at[pl.ds(base, G)], S0)
        pltpu.sync_copy(b_hbm.at[pl.ds(base, G)], T0)
        _sort_rows(S0, S1)
        _sort_rows(T0, T1)
        rs = _row_sums(S0, T0)
        return loss + _vsqrt(rs * (1.0 / N))

    loss = lax.fori_loop(0, NGRP, group, jnp.zeros((G,), jnp.float32))
    accv[...] = loss
    pltpu.sync_copy(accv, out_hbm.at[wid])


def kernel(hidden_states, hidden_states_aug):
    a = hidden_states.reshape(ROWS, N)
    b = hidden_states_aug.reshape(ROWS, N)
    out = _sc_loss(a, b)
    return jnp.sum(out) * jnp.float32(1.0 / ROWS)


# speedup vs baseline: 3.9635x; 3.9635x over previous
import functools

import jax
import jax.numpy as jnp
from jax import lax
from jax.experimental import pallas as pl
from jax.experimental.pallas import tpu as pltpu
from jax.experimental.pallas import tpu_sc as plsc

NC, NS = 2, 16
NW = NC * NS
ROWS = 16384
N = 1024
RPW = ROWS // NW
G = 16
NGRP = RPW // G
CHUNKS = N // 16


def _vsqrt(x):
    i = lax.bitcast_convert_type(x, jnp.int32)
    y = lax.bitcast_convert_type((i >> 1) + jnp.int32(0x1FBD1DF6), jnp.float32)
    for _ in range(3):
        y = 0.5 * (y + x / y)
    return y


def _sort16(v):
    return lax.sort(v, dimension=0, is_stable=False)


def _sort_groups(S0, S1, T0, T1):
    @plsc.parallel_loop(0, G * CHUNKS, unroll=2)
    def _(u):
        j = u >> 6
        off = (u & (CHUNKS - 1)) * 16
        S0[j, pl.ds(off, 16)] = _sort16(jnp.abs(S0[j, pl.ds(off, 16)]))
        T0[j, pl.ds(off, 16)] = _sort16(jnp.abs(T0[j, pl.ds(off, 16)]))

    bufs = ((S0, T0), (S1, T1))
    cur = 0
    for m in (16, 32, 64, 128, 256, 512):
        (Ss, Ts), (Sd, Td) = bufs[cur], bufs[1 - cur]
        k = m // 16
        kb = k.bit_length() - 1

        @plsc.parallel_loop(0, G * 32, unroll=2)
        def _(u, m=m, k=k, kb=kb, Ss=Ss, Ts=Ts, Sd=Sd, Td=Td):
            j = u >> 5
            t = u & 31
            mi = t >> kb
            jj = t & (k - 1)
            base = mi * (2 * m)
            lo = base + jj * 16
            hs = base + 2 * m - (jj + 1) * 16
            hd = base + m + jj * 16
            for A, B in ((Ss, Sd), (Ts, Td)):
                va = A[j, pl.ds(lo, 16)]
                vb = lax.rev(A[j, pl.ds(hs, 16)], (0,))
                B[j, pl.ds(lo, 16)] = jnp.minimum(va, vb)
                B[j, pl.ds(hd, 16)] = jnp.maximum(va, vb)

        d = m // 2
        while d >= 16:
            dk = d // 16
            dkb = dk.bit_length() - 1

            @plsc.parallel_loop(0, G * 32, unroll=2)
            def _(u, m=m, d=d, k=k, kb=kb, dk=dk, dkb=dkb, Sd=Sd, Td=Td):
                j = u >> 5
                t = u & 31
                mi = t >> kb
                tl = t & (k - 1)
                b = tl >> dkb
                jj = tl & (dk - 1)
                off = mi * (2 * m) + b * (2 * d) + jj * 16
                for A in (Sd, Td):
                    xlo = A[j, pl.ds(off, 16)]
                    xhi = A[j, pl.ds(off + d, 16)]
                    A[j, pl.ds(off, 16)] = jnp.minimum(xlo, xhi)
                    A[j, pl.ds(off + d, 16)] = jnp.maximum(xlo, xhi)

            d //= 2

        @plsc.parallel_loop(0, G * CHUNKS, unroll=2)
        def _(u, Sd=Sd, Td=Td):
            j = u >> 6
            off = (u & (CHUNKS - 1)) * 16
            for A in (Sd, Td):
                A[j, pl.ds(off, 16)] = _sort16(A[j, pl.ds(off, 16)])

        cur = 1 - cur


def _row_sums(S0, T0):
    rows = lax.iota(jnp.int32, 16)

    @plsc.parallel_loop(0, N, unroll=4, carry=jnp.zeros((16,), jnp.float32))
    def acc_loop(p, acc):
        col = jnp.full((16,), p, dtype=jnp.int32)
        av = plsc.load_gather(S0, [rows, col])
        bv = plsc.load_gather(T0, [rows, col])
        dd = av - bv
        return acc + dd * dd

    return acc_loop


_MESH = plsc.VectorSubcoreMesh(
    core_axis_name="c", subcore_axis_name="s", num_cores=NC, num_subcores=NS
)


@functools.partial(
    pl.kernel,
    mesh=_MESH,
    out_type=jax.ShapeDtypeStruct((NW, G), jnp.float32),
    compiler_params=pltpu.CompilerParams(needs_layout_passes=False),
    scratch_types=[
        pltpu.VMEM((G, N), jnp.float32),
        pltpu.VMEM((G, N), jnp.float32),
        pltpu.VMEM((G, N), jnp.float32),
        pltpu.VMEM((G, N), jnp.float32),
        pltpu.VMEM((G,), jnp.float32),
    ],
)
def _sc_loss(a_hbm, b_hbm, out_hbm, S0, S1, T0, T1, accv):
    wid = lax.axis_index("s") * NC + lax.axis_index("c")

    def group(t, loss):
        base = wid * RPW + t * G
        pltpu.sync_copy(a_hbm.at[pl.ds(base, G)], S0)
        pltpu.sync_copy(b_hbm.at[pl.ds(base, G)], T0)
        _sort_groups(S0, S1, T0, T1)
        rs = _row_sums(S0, T0)
        return loss + _vsqrt(rs * (1.0 / N))

    loss = lax.fori_loop(0, NGRP, group, jnp.zeros((G,), jnp.float32))
    accv[...] = loss
    pltpu.sync_copy(accv, out_hbm.at[wid])


def kernel(hidden_states, hidden_states_aug):
    a = hidden_states.reshape(ROWS, N)
    b = hidden_states_aug.reshape(ROWS, N)
    out = _sc_loss(a, b)
    return jnp.sum(out) * jnp.float32(1.0 / ROWS)


# speedup vs baseline: 6.5891x; 1.6624x over previous
import functools

import jax
import jax.numpy as jnp
from jax import lax
from jax.experimental import pallas as pl
from jax.experimental.pallas import tpu as pltpu
from jax.experimental.pallas import tpu_sc as plsc

NC, NS = 2, 16
NW = NC * NS
ROWS = 16384
N = 1024
RPW = ROWS // NW
G = 16
NGRP = RPW // G


def _vsqrt(x):
    i = lax.bitcast_convert_type(x, jnp.int32)
    y = lax.bitcast_convert_type((i >> 1) + jnp.int32(0x1FBD1DF6), jnp.float32)
    for _ in range(3):
        y = 0.5 * (y + x / y)
    return y


def _sort16(v):
    return lax.sort(v, dimension=0, is_stable=False)


def _refine(v, dv0):
    v = list(v)
    dv = dv0
    while dv >= 1:
        for b in range(0, len(v), 2 * dv):
            for t in range(dv):
                x, y = v[b + t], v[b + dv + t]
                v[b + t] = jnp.minimum(x, y)
                v[b + dv + t] = jnp.maximum(x, y)
        dv //= 2
    return [_sort16(x) for x in v]


def _block256_sort(A, j, cbase):
    v = [_sort16(jnp.abs(A[j, pl.ds(cbase + i * 16, 16)])) for i in range(16)]
    for r in (1, 2, 4, 8):
        nv = []
        for s in range(0, 16, 2 * r):
            lo, hi = [], []
            for jj in range(r):
                va = v[s + jj]
                vb = lax.rev(v[s + 2 * r - 1 - jj], (0,))
                lo.append(jnp.minimum(va, vb))
                hi.append(jnp.maximum(va, vb))
            nv.extend(_refine(lo + hi, r // 2))
        v = nv
    for i in range(16):
        A[j, pl.ds(cbase + i * 16, 16)] = v[i]


def _fused_tail(A, j, cbase):
    v = [A[j, pl.ds(cbase + i * 16, 16)] for i in range(16)]
    v = _refine(v, 8)
    for i in range(16):
        A[j, pl.ds(cbase + i * 16, 16)] = v[i]


def _sort_groups(S0, S1, T0, T1):
    @plsc.parallel_loop(0, G * 4, unroll=1)
    def _(u):
        j = u >> 2
        cbase = (u & 3) * 256
        _block256_sort(S0, j, cbase)
        _block256_sort(T0, j, cbase)

    @plsc.parallel_loop(0, G * 32, unroll=2)
    def _(u):
        j = u >> 5
        t = u & 31
        base = (t >> 4) * 512
        jj = t & 15
        lo = base + jj * 16
        hs = base + 512 - (jj + 1) * 16
        hd = base + 256 + jj * 16
        for A, B in ((S0, S1), (T0, T1)):
            va = A[j, pl.ds(lo, 16)]
            vb = lax.rev(A[j, pl.ds(hs, 16)], (0,))
            B[j, pl.ds(lo, 16)] = jnp.minimum(va, vb)
            B[j, pl.ds(hd, 16)] = jnp.maximum(va, vb)

    @plsc.parallel_loop(0, G * 4, unroll=1)
    def _(u):
        j = u >> 2
        cbase = (u & 3) * 256
        _fused_tail(S1, j, cbase)
        _fused_tail(T1, j, cbase)

    @plsc.parallel_loop(0, G * 32, unroll=2)
    def _(u):
        j = u >> 5
        jj = u & 31
        lo = jj * 16
        hs = N - (jj + 1) * 16
        hd = 512 + jj * 16
        for A, B in ((S1, S0), (T1, T0)):
            va = A[j, pl.ds(lo, 16)]
            vb = lax.rev(A[j, pl.ds(hs, 16)], (0,))
            B[j, pl.ds(lo, 16)] = jnp.minimum(va, vb)
            B[j, pl.ds(hd, 16)] = jnp.maximum(va, vb)

    @plsc.parallel_loop(0, G * 32, unroll=2)
    def _(u):
        j = u >> 5
        t = u & 31
        off = (t >> 4) * 512 + (t & 15) * 16
        for A in (S0, T0):
            x = A[j, pl.ds(off, 16)]
            y = A[j, pl.ds(off + 256, 16)]
            A[j, pl.ds(off, 16)] = jnp.minimum(x, y)
            A[j, pl.ds(off + 256, 16)] = jnp.maximum(x, y)

    @plsc.parallel_loop(0, G * 4, unroll=1)
    def _(u):
        j = u >> 2
        cbase = (u & 3) * 256
        _fused_tail(S0, j, cbase)
        _fused_tail(T0, j, cbase)


def _row_sums(S0, T0):
    rows = lax.iota(jnp.int32, 16)

    @plsc.parallel_loop(0, N, unroll=8, carry=jnp.zeros((16,), jnp.float32))
    def acc_loop(p, acc):
        col = jnp.full((16,), p, dtype=jnp.int32)
        av = plsc.load_gather(S0, [rows, col])
        bv = plsc.load_gather(T0, [rows, col])
        dd = av - bv
        return acc + dd * dd

    return acc_loop


_MESH = plsc.VectorSubcoreMesh(
    core_axis_name="c", subcore_axis_name="s", num_cores=NC, num_subcores=NS
)


@functools.partial(
    pl.kernel,
    mesh=_MESH,
    out_type=jax.ShapeDtypeStruct((NW, G), jnp.float32),
    compiler_params=pltpu.CompilerParams(needs_layout_passes=False),
    scratch_types=[
        pltpu.VMEM((G, N), jnp.float32),
        pltpu.VMEM((G, N), jnp.float32),
        pltpu.VMEM((G, N), jnp.float32),
        pltpu.VMEM((G, N), jnp.float32),
        pltpu.VMEM((G,), jnp.float32),
    ],
)
def _sc_loss(a_hbm, b_hbm, out_hbm, S0, S1, T0, T1, accv):
    wid = lax.axis_index("s") * NC + lax.axis_index("c")

    def group(t, loss):
        base = wid * RPW + t * G
        pltpu.sync_copy(a_hbm.at[pl.ds(base, G)], S0)
        pltpu.sync_copy(b_hbm.at[pl.ds(base, G)], T0)
        _sort_groups(S0, S1, T0, T1)
        rs = _row_sums(S0, T0)
        return loss + _vsqrt(rs * (1.0 / N))

    loss = lax.fori_loop(0, NGRP, group, jnp.zeros((G,), jnp.float32))
    accv[...] = loss
    pltpu.sync_copy(accv, out_hbm.at[wid])


def kernel(hidden_states, hidden_states_aug):
    a = hidden_states.reshape(ROWS, N)
    b = hidden_states_aug.reshape(ROWS, N)
    out = _sc_loss(a, b)
    return jnp.sum(out) * jnp.float32(1.0 / ROWS)


# speedup vs baseline: 7.2099x; 1.0942x over previous
import functools

import jax
import jax.numpy as jnp
from jax import lax
from jax.experimental import pallas as pl
from jax.experimental.pallas import tpu as pltpu
from jax.experimental.pallas import tpu_sc as plsc

NC, NS = 2, 16
NW = NC * NS
ROWS = 16384
N = 1024
RPW = ROWS // NW
G = 16
NGRP = RPW // G


def _vsqrt(x):
    i = lax.bitcast_convert_type(x, jnp.int32)
    y = lax.bitcast_convert_type((i >> 1) + jnp.int32(0x1FBD1DF6), jnp.float32)
    for _ in range(3):
        y = 0.5 * (y + x / y)
    return y


def _sort16(v, asc):
    return plsc.sort_key_val(v, v, descending=not asc)[0]


def _refine(v, asc):
    v = list(v)
    n = len(v)
    dv = n // 2
    while dv >= 1:
        for b in range(0, n, 2 * dv):
            for t in range(dv):
                x, y = v[b + t], v[b + dv + t]
                lo, hi = jnp.minimum(x, y), jnp.maximum(x, y)
                v[b + t], v[b + dv + t] = (lo, hi) if asc else (hi, lo)
        dv //= 2
    return [_sort16(x, asc) for x in v]


def _bsort(v, asc):
    if len(v) == 1:
        return [_sort16(v[0], asc)]
    h = len(v) // 2
    a = _bsort(v[:h], asc)
    b = _bsort(v[h:], not asc)
    return _refine(a + b, asc)


def _block256(A, j, cbase, asc):
    v = [jnp.abs(A[j, pl.ds(cbase + i * 16, 16)]) for i in range(16)]
    v = _bsort(v, asc)
    for i in range(16):
        A[j, pl.ds(cbase + i * 16, 16)] = v[i]


def _refine512(A, j, cbase, asc):
    v = [A[j, pl.ds(cbase + i * 16, 16)] for i in range(32)]
    v = _refine(v, asc)
    for i in range(32):
        A[j, pl.ds(cbase + i * 16, 16)] = v[i]


def _sort_groups(S0, T0):
    for par in (0, 1):
        @plsc.parallel_loop(0, G * 2, unroll=1)
        def _(u, par=par):
            j = u >> 1
            cbase = ((u & 1) * 2 + par) * 256
            _block256(S0, j, cbase, par == 0)
            _block256(T0, j, cbase, par == 0)

    for A in (S0, T0):
        for half, asc in ((0, True), (1, False)):
            @plsc.parallel_loop(0, G, unroll=1)
            def _(j, A=A, half=half, asc=asc):
                _refine512(A, j, half * 512, asc)

    @plsc.parallel_loop(0, G * 32, unroll=4)
    def _(u):
        j = u >> 5
        off = (u & 31) * 16
        for A in (S0, T0):
            x = A[j, pl.ds(off, 16)]
            y = A[j, pl.ds(off + 512, 16)]
            A[j, pl.ds(off, 16)] = jnp.minimum(x, y)
            A[j, pl.ds(off + 512, 16)] = jnp.maximum(x, y)

    for A in (S0, T0):
        @plsc.parallel_loop(0, G * 2, unroll=1)
        def _(u, A=A):
            _refine512(A, u >> 1, (u & 1) * 512, True)


def _row_sums(S0, T0):
    rows = lax.iota(jnp.int32, 16)

    @plsc.parallel_loop(0, N, unroll=8, carry=jnp.zeros((16,), jnp.float32))
    def acc_loop(p, acc):
        col = jnp.full((16,), p, dtype=jnp.int32)
        av = plsc.load_gather(S0, [rows, col])
        bv = plsc.load_gather(T0, [rows, col])
        dd = av - bv
        return acc + dd * dd

    return acc_loop


_MESH = plsc.VectorSubcoreMesh(
    core_axis_name="c", subcore_axis_name="s", num_cores=NC, num_subcores=NS
)


@functools.partial(
    pl.kernel,
    mesh=_MESH,
    out_type=jax.ShapeDtypeStruct((NW, G), jnp.float32),
    compiler_params=pltpu.CompilerParams(needs_layout_passes=False),
    scratch_types=[
        pltpu.VMEM((G, N), jnp.float32),
        pltpu.VMEM((G, N), jnp.float32),
        pltpu.VMEM((G,), jnp.float32),
    ],
)
def _sc_loss(a_hbm, b_hbm, out_hbm, S0, T0, accv):
    wid = lax.axis_index("s") * NC + lax.axis_index("c")

    def group(t, loss):
        base = wid * RPW + t * G
        pltpu.sync_copy(a_hbm.at[pl.ds(base, G)], S0)
        pltpu.sync_copy(b_hbm.at[pl.ds(base, G)], T0)
        _sort_groups(S0, T0)
        rs = _row_sums(S0, T0)
        return loss + _vsqrt(rs * (1.0 / N))

    loss = lax.fori_loop(0, NGRP, group, jnp.zeros((G,), jnp.float32))
    accv[...] = loss
    pltpu.sync_copy(accv, out_hbm.at[wid])


def kernel(hidden_states, hidden_states_aug):
    a = hidden_states.reshape(ROWS, N)
    b = hidden_states_aug.reshape(ROWS, N)
    out = _sc_loss(a, b)
    return jnp.sum(out) * jnp.float32(1.0 / ROWS)


# speedup vs baseline: 13.2682x; 1.8403x over previous
import functools

import jax
import jax.numpy as jnp
from jax import lax
from jax.experimental import pallas as pl
from jax.experimental.pallas import tpu as pltpu
from jax.experimental.pallas import tpu_sc as plsc

NC, NS = 2, 16
NW = NC * NS
ROWS = 16384
N = 1024
RPW = ROWS // NW
G = 16
NGRP = RPW // G


def _vsqrt(x):
    i = lax.bitcast_convert_type(x, jnp.int32)
    y = lax.bitcast_convert_type((i >> 1) + jnp.int32(0x1FBD1DF6), jnp.float32)
    for _ in range(3):
        y = 0.5 * (y + x / y)
    return y


def _sort16(v, asc):
    return plsc.sort_key_val(v, v, descending=not asc)[0]


def _refine(v, asc):
    v = list(v)
    n = len(v)
    dv = n // 2
    while dv >= 1:
        for b in range(0, n, 2 * dv):
            for t in range(dv):
                x, y = v[b + t], v[b + dv + t]
                lo, hi = jnp.minimum(x, y), jnp.maximum(x, y)
                v[b + t], v[b + dv + t] = (lo, hi) if asc else (hi, lo)
        dv //= 2
    return [_sort16(x, asc) for x in v]


def _bsort(v, asc):
    if len(v) == 1:
        return [_sort16(v[0], asc)]
    h = len(v) // 2
    a = _bsort(v[:h], asc)
    b = _bsort(v[h:], not asc)
    return _refine(a + b, asc)


def _block512(A, j, cbase, asc):
    v = [jnp.abs(A[j, pl.ds(cbase + i * 16, 16)]) for i in range(32)]
    v = _bsort(v, asc)
    for i in range(32):
        A[j, pl.ds(cbase + i * 16, 16)] = v[i]


def _sort_and_reduce(S0, T0, R):
    for A in (S0, T0):
        for half, asc in ((0, True), (1, False)):
            @plsc.parallel_loop(0, G, unroll=1)
            def _(j, A=A, half=half, asc=asc):
                _block512(A, j, half * 512, asc)

    @plsc.parallel_loop(0, G * 32, unroll=4)
    def _(u):
        j = u >> 5
        off = (u & 31) * 16
        for A in (S0, T0):
            x = A[j, pl.ds(off, 16)]
            y = A[j, pl.ds(off + 512, 16)]
            A[j, pl.ds(off, 16)] = jnp.minimum(x, y)
            A[j, pl.ds(off + 512, 16)] = jnp.maximum(x, y)

    @plsc.parallel_loop(0, G * 2, unroll=1)
    def _(u):
        j = u >> 1
        cbase = (u & 1) * 512
        v = [S0[j, pl.ds(cbase + i * 16, 16)] for i in range(32)]
        v = _refine(v, True)
        for i in range(32):
            S0[j, pl.ds(cbase + i * 16, 16)] = v[i]

    @plsc.parallel_loop(0, G, unroll=1)
    def _(j):
        acc = jnp.zeros((16,), jnp.float32)
        for half in (0, 1):
            cbase = half * 512
            v = [T0[j, pl.ds(cbase + i * 16, 16)] for i in range(32)]
            v = _refine(v, True)
            for i in range(32):
                dd = S0[j, pl.ds(cbase + i * 16, 16)] - v[i]
                acc = acc + dd * dd
        R[j, :] = acc


def _row_sums(R):
    rows = lax.iota(jnp.int32, 16)
    acc = jnp.zeros((16,), jnp.float32)
    for p in range(16):
        col = jnp.full((16,), p, dtype=jnp.int32)
        acc = acc + plsc.load_gather(R, [rows, col])
    return acc


_MESH = plsc.VectorSubcoreMesh(
    core_axis_name="c", subcore_axis_name="s", num_cores=NC, num_subcores=NS
)


@functools.partial(
    pl.kernel,
    mesh=_MESH,
    out_type=jax.ShapeDtypeStruct((NW, G), jnp.float32),
    compiler_params=pltpu.CompilerParams(needs_layout_passes=False),
    scratch_types=[
        pltpu.VMEM((G, N), jnp.float32),
        pltpu.VMEM((G, N), jnp.float32),
        pltpu.VMEM((G, G), jnp.float32),
        pltpu.VMEM((G,), jnp.float32),
    ],
)
def _sc_loss(a_hbm, b_hbm, out_hbm, S0, T0, R, accv):
    wid = lax.axis_index("s") * NC + lax.axis_index("c")

    def group(t, loss):
        base = wid * RPW + t * G
        pltpu.sync_copy(a_hbm.at[pl.ds(base, G)], S0)
        pltpu.sync_copy(b_hbm.at[pl.ds(base, G)], T0)
        _sort_and_reduce(S0, T0, R)
        rs = _row_sums(R)
        return loss + _vsqrt(rs * (1.0 / N))

    loss = lax.fori_loop(0, NGRP, group, jnp.zeros((G,), jnp.float32))
    accv[...] = loss
    pltpu.sync_copy(accv, out_hbm.at[wid])


def kernel(hidden_states, hidden_states_aug):
    a = hidden_states.reshape(ROWS, N)
    b = hidden_states_aug.reshape(ROWS, N)
    out = _sc_loss(a, b)
    return jnp.sum(out) * jnp.float32(1.0 / ROWS)


# speedup vs baseline: 14.8842x; 1.1218x over previous
import functools

import jax
import jax.numpy as jnp
from jax import lax
from jax.experimental import pallas as pl
from jax.experimental.pallas import tpu as pltpu
from jax.experimental.pallas import tpu_sc as plsc

NC, NS = 2, 16
NW = NC * NS
ROWS = 16384
N = 1024
RPW = ROWS // NW
G = 16
NGRP = RPW // G


def _vsqrt(x):
    i = lax.bitcast_convert_type(x, jnp.int32)
    y = lax.bitcast_convert_type((i >> 1) + jnp.int32(0x1FBD1DF6), jnp.float32)
    for _ in range(3):
        y = 0.5 * (y + x / y)
    return y


def _sort16(v, asc):
    return plsc.sort_key_val(v, v, descending=not asc)[0]


def _refine(v, asc):
    v = list(v)
    n = len(v)
    dv = n // 2
    while dv >= 1:
        for b in range(0, n, 2 * dv):
            for t in range(dv):
                x, y = v[b + t], v[b + dv + t]
                lo, hi = jnp.minimum(x, y), jnp.maximum(x, y)
                v[b + t], v[b + dv + t] = (lo, hi) if asc else (hi, lo)
        dv //= 2
    return [_sort16(x, asc) for x in v]


def _bsort(v, asc):
    if len(v) == 1:
        return [_sort16(v[0], asc)]
    h = len(v) // 2
    a = _bsort(v[:h], asc)
    b = _bsort(v[h:], not asc)
    return _refine(a + b, asc)


def _block512(A, j, cbase, asc):
    v = [jnp.abs(A[j, pl.ds(cbase + i * 16, 16)]) for i in range(32)]
    v = _bsort(v, asc)
    for i in range(32):
        A[j, pl.ds(cbase + i * 16, 16)] = v[i]


def _sort_and_reduce(S0, T0, R):
    for A in (S0, T0):
        for half, asc in ((0, True), (1, False)):
            @plsc.parallel_loop(0, G, unroll=1)
            def _(j, A=A, half=half, asc=asc):
                _block512(A, j, half * 512, asc)

    @plsc.parallel_loop(0, G * 32, unroll=4)
    def _(u):
        j = u >> 5
        off = (u & 31) * 16
        for A in (S0, T0):
            x = A[j, pl.ds(off, 16)]
            y = A[j, pl.ds(off + 512, 16)]
            A[j, pl.ds(off, 16)] = jnp.minimum(x, y)
            A[j, pl.ds(off + 512, 16)] = jnp.maximum(x, y)

    @plsc.parallel_loop(0, G * 2, unroll=1)
    def _(u):
        j = u >> 1
        cbase = (u & 1) * 512
        v = [S0[j, pl.ds(cbase + i * 16, 16)] for i in range(32)]
        v = _refine(v, True)
        for i in range(32):
            S0[j, pl.ds(cbase + i * 16, 16)] = v[i]

    @plsc.parallel_loop(0, G, unroll=1)
    def _(j):
        acc = jnp.zeros((16,), jnp.float32)
        for half in (0, 1):
            cbase = half * 512
            v = [T0[j, pl.ds(cbase + i * 16, 16)] for i in range(32)]
            v = _refine(v, True)
            for i in range(32):
                dd = S0[j, pl.ds(cbase + i * 16, 16)] - v[i]
                acc = acc + dd * dd
        R[j, :] = acc


def _row_sums(R):
    rows = lax.iota(jnp.int32, 16)
    acc = jnp.zeros((16,), jnp.float32)
    for p in range(16):
        col = jnp.full((16,), p, dtype=jnp.int32)
        acc = acc + plsc.load_gather(R, [rows, col])
    return acc


_MESH = plsc.VectorSubcoreMesh(
    core_axis_name="c", subcore_axis_name="s", num_cores=NC, num_subcores=NS
)


@functools.partial(
    pl.kernel,
    mesh=_MESH,
    out_type=jax.ShapeDtypeStruct((NW, G), jnp.float32),
    compiler_params=pltpu.CompilerParams(needs_layout_passes=False),
    scratch_types=[
        pltpu.VMEM((G, N), jnp.float32),
        pltpu.VMEM((G, N), jnp.float32),
        pltpu.VMEM((G, N), jnp.float32),
        pltpu.VMEM((G, N), jnp.float32),
        pltpu.VMEM((G, G), jnp.float32),
        pltpu.VMEM((G,), jnp.float32),
        pltpu.SemaphoreType.DMA,
        pltpu.SemaphoreType.DMA,
    ],
)
def _sc_loss(a_hbm, b_hbm, out_hbm, S0, T0, S1, T1, R, accv, sem0, sem1):
    wid = lax.axis_index("s") * NC + lax.axis_index("c")

    def start(t, S, T, sem):
        base = wid * RPW + t * G
        pltpu.async_copy(a_hbm.at[pl.ds(base, G)], S, sem)
        pltpu.async_copy(b_hbm.at[pl.ds(base, G)], T, sem)

    def wait(S, T, sem):
        pltpu.make_async_copy(a_hbm.at[pl.ds(0, G)], S, sem).wait()
        pltpu.make_async_copy(b_hbm.at[pl.ds(0, G)], T, sem).wait()

    def compute(S, T, loss):
        _sort_and_reduce(S, T, R)
        rs = _row_sums(R)
        return loss + _vsqrt(rs * (1.0 / N))

    start(0, S0, T0, sem0)

    def group2(t2, loss):
        start(2 * t2 + 1, S1, T1, sem1)
        wait(S0, T0, sem0)
        loss = compute(S0, T0, loss)

        @pl.when(t2 < NGRP // 2 - 1)
        def _():
            start(2 * t2 + 2, S0, T0, sem0)

        wait(S1, T1, sem1)
        return compute(S1, T1, loss)

    loss = lax.fori_loop(0, NGRP // 2, group2, jnp.zeros((G,), jnp.float32))
    accv[...] = loss
    pltpu.sync_copy(accv, out_hbm.at[wid])


def kernel(hidden_states, hidden_states_aug):
    a = hidden_states.reshape(ROWS, N)
    b = hidden_states_aug.reshape(ROWS, N)
    out = _sc_loss(a, b)
    return jnp.sum(out) * jnp.float32(1.0 / ROWS)
